# Initial kernel scaffold; baseline (speedup 1.0000x reference)
import jax
import jax.numpy as jnp
from jax.experimental import pallas as pl


def kernel(x, edge_index, batch, W0, b0, W1, b1, W2, b2, W3, b3, conv1_w, conv1_b, conv2_w, conv2_b, mlp_w1, mlp_b1, mlp_w2, mlp_b2):
    raise NotImplementedError("write your pallas kernel here")


# speedup vs baseline: 6.8136x; 6.8136x over previous
import functools

import numpy as np
import jax
import jax.numpy as jnp
from jax import lax
from jax.experimental import pallas as pl
from jax.experimental.pallas import tpu as pltpu
from jax.experimental.pallas import tpu_sc as plsc

N = 10000
E = 320000
IN_CH = 128
H = 32
B = 64
K = 30

NC = 2
NS = 16
NW = NC * NS
NPAD = 10240
SLAB = NPAD // NS
RNG = NPAD // NW
ACC = RNG + 8
CH = 128
SCH = 2560
NSCHUNK = E // SCH
BUF = 2048
ESUB = 158 * BUF
EP = 323584
EPW = EP // NW
NCH_DEG = EPW // CH
NB = 10112
NI = 2048
IPW = NI // NW
RB = 1000
GB = N // RB

_MESH = plsc.VectorSubcoreMesh(core_axis_name="c", subcore_axis_name="s")
_SCPARAMS = pltpu.CompilerParams(use_tc_tiling_on_sc=False,
                                 needs_layout_passes=False)


@functools.partial(
    pl.kernel,
    out_type=jax.ShapeDtypeStruct((NC * NPAD, 16), jnp.float32),
    mesh=_MESH,
    compiler_params=_SCPARAMS,
    scratch_types=[
        pltpu.VMEM((CH,), jnp.int32),
        pltpu.VMEM((CH, 16), jnp.float32),
        pltpu.VMEM_SHARED((NPAD, 16), jnp.float32),
    ],
)
def _sc_deg(dst_hbm, ones_hbm, zeros_hbm, out_hbm, dst_v, ones_v, acc_sh):
    c = lax.axis_index("c")
    s = lax.axis_index("s")
    wid = s * NC + c
    pltpu.sync_copy(ones_hbm, ones_v)
    pltpu.sync_copy(zeros_hbm.at[pl.ds(s * SLAB, SLAB)],
                    acc_sh.at[pl.ds(s * SLAB, SLAB)])
    plsc.subcore_barrier()
    base = wid * EPW

    def body(i, carry):
        off = base + i * CH
        pltpu.sync_copy(dst_hbm.at[pl.ds(off, CH)], dst_v)
        pltpu.sync_copy(ones_v, acc_sh.at[dst_v], add=True)
        return carry

    lax.fori_loop(0, NCH_DEG, body, 0)
    plsc.subcore_barrier()
    pltpu.sync_copy(acc_sh.at[pl.ds(s * SLAB, SLAB)],
                    out_hbm.at[pl.ds(c * NPAD + s * SLAB, SLAB)])


@functools.partial(
    pl.kernel,
    out_type=[
        jax.ShapeDtypeStruct((NW * ESUB,), jnp.int32),
        jax.ShapeDtypeStruct((NW * ESUB,), jnp.int32),
        jax.ShapeDtypeStruct((NW * ESUB,), jnp.float32),
        jax.ShapeDtypeStruct((NW * 16,), jnp.int32),
    ],
    mesh=_MESH,
    compiler_params=_SCPARAMS,
    scratch_types=[
        pltpu.VMEM((SCH,), jnp.int32),
        pltpu.VMEM((SCH,), jnp.int32),
        pltpu.VMEM((NPAD,), jnp.float32),
        pltpu.VMEM((BUF + 128,), jnp.int32),
        pltpu.VMEM((BUF + 128,), jnp.int32),
        pltpu.VMEM((BUF + 128,), jnp.float32),
        pltpu.VMEM((16,), jnp.int32),
    ],
)
def _sc_setup(src_hbm, dst_hbm, dinv_hbm,
              srcg_hbm, dloc_hbm, nrm_hbm, cnt_hbm,
              src_v, dst_v, dinv_v, sbuf, dbuf, nbuf, cnt_v):
    c = lax.axis_index("c")
    s = lax.axis_index("s")
    wid = s * NC + c
    lo = wid * RNG
    obase = wid * ESUB
    pltpu.sync_copy(dinv_hbm, dinv_v)
    iota = lax.iota(jnp.int32, 16)

    def flush(off, nblk):
        full = off >= BUF

        @pl.when(full)
        def _():
            dst0 = obase + nblk * BUF
            pltpu.sync_copy(sbuf.at[pl.ds(0, BUF)], srcg_hbm.at[pl.ds(dst0, BUF)])
            pltpu.sync_copy(dbuf.at[pl.ds(0, BUF)], dloc_hbm.at[pl.ds(dst0, BUF)])
            pltpu.sync_copy(nbuf.at[pl.ds(0, BUF)], nrm_hbm.at[pl.ds(dst0, BUF)])
            for t in range(8):
                sbuf[pl.ds(t * 16, 16)] = sbuf[pl.ds(BUF + t * 16, 16)]
                dbuf[pl.ds(t * 16, 16)] = dbuf[pl.ds(BUF + t * 16, 16)]
                nbuf[pl.ds(t * 16, 16)] = nbuf[pl.ds(BUF + t * 16, 16)]

        return (jnp.where(full, off - BUF, off),
                jnp.where(full, nblk + 1, nblk))

    def chunk(i, carry):
        off, nblk = carry
        pltpu.sync_copy(src_hbm.at[pl.ds(i * SCH, SCH)], src_v)
        pltpu.sync_copy(dst_hbm.at[pl.ds(i * SCH, SCH)], dst_v)
        for j in range(SCH // 16):
            s16 = src_v[pl.ds(j * 16, 16)]
            d16 = dst_v[pl.ds(j * 16, 16)]
            m = (d16 >= lo) & (d16 < lo + RNG)
            n16 = plsc.load_gather(dinv_v, [s16]) * plsc.load_gather(dinv_v, [d16])
            plsc.store_compressed(sbuf.at[pl.ds(off, 16)], s16, mask=m)
            plsc.store_compressed(dbuf.at[pl.ds(off, 16)], d16 - lo, mask=m)
            plsc.store_compressed(nbuf.at[pl.ds(off, 16)], n16, mask=m)
            off = off + plsc.all_reduce_population_count(m)[0]
            if j % 8 == 7:
                off, nblk = flush(off, nblk)
        return off, nblk

    off, nblk = lax.fori_loop(0, NSCHUNK, chunk, (jnp.int32(0), jnp.int32(0)))
    off, nblk = flush(off, nblk)

    def padslots(t, carry):
        base16 = t * 16
        mfill = (base16 + iota) >= off
        sbuf[pl.ds(base16, 16)] = jnp.where(mfill, N, sbuf[pl.ds(base16, 16)])
        dbuf[pl.ds(base16, 16)] = jnp.where(mfill, RNG, dbuf[pl.ds(base16, 16)])
        nbuf[pl.ds(base16, 16)] = jnp.where(
            mfill, jnp.float32(0.0), nbuf[pl.ds(base16, 16)])
        return carry

    lax.fori_loop(0, BUF // 16, padslots, 0)
    dst0 = obase + nblk * BUF
    pltpu.sync_copy(sbuf.at[pl.ds(0, BUF)], srcg_hbm.at[pl.ds(dst0, BUF)])
    pltpu.sync_copy(dbuf.at[pl.ds(0, BUF)], dloc_hbm.at[pl.ds(dst0, BUF)])
    pltpu.sync_copy(nbuf.at[pl.ds(0, BUF)], nrm_hbm.at[pl.ds(dst0, BUF)])
    cnt_v[...] = jnp.zeros((16,), jnp.int32) + (nblk * BUF + off)
    pltpu.sync_copy(cnt_v, cnt_hbm.at[pl.ds(wid * 16, 16)])


SUP = 1024


def _make_sc_agg(d):
    @functools.partial(
        pl.kernel,
        out_type=jax.ShapeDtypeStruct((NPAD, d), jnp.float32),
        mesh=_MESH,
        compiler_params=_SCPARAMS,
        scratch_types=[
            pltpu.VMEM((SUP,), jnp.int32),
            pltpu.VMEM((SUP,), jnp.int32),
            pltpu.VMEM((SUP,), jnp.float32),
            pltpu.VMEM((SUP, d), jnp.float32),
            pltpu.VMEM((ACC, d), jnp.float32),
            pltpu.VMEM((16,), jnp.int32),
            pltpu.SemaphoreType.DMA,
        ],
    )
    def agg(h_hbm, srcg_hbm, dloc_hbm, nrm_hbm, cnt_hbm, zeros_hbm, out_hbm,
            idx_v, dl_v, nr_v, rows_v, acc_v, cnt_v, sem):
        c = lax.axis_index("c")
        s = lax.axis_index("s")
        wid = s * NC + c
        obase = wid * ESUB
        pltpu.sync_copy(cnt_hbm.at[pl.ds(wid * 16, 16)], cnt_v)
        pltpu.sync_copy(zeros_hbm, acc_v)
        cnt = cnt_v[...][0]
        nch = lax.div(cnt + (SUP - 1), SUP)
        iota = lax.iota(jnp.int32, 16)

        def chunk(i, carry):
            o = obase + i * SUP
            pltpu.sync_copy(srcg_hbm.at[pl.ds(o, SUP)], idx_v)
            pltpu.sync_copy(dloc_hbm.at[pl.ds(o, SUP)], dl_v)
            pltpu.sync_copy(nrm_hbm.at[pl.ds(o, SUP)], nr_v)
            descs = [
                pltpu.async_copy(
                    h_hbm.at[idx_v.at[pl.ds(k * CH, CH)]],
                    rows_v.at[pl.ds(k * CH, CH)], sem)
                for k in range(SUP // CH)
            ]
            for dsc in descs:
                dsc.wait()

            def grp(g, carry2):
                dl16 = dl_v[pl.ds(g * 16, 16)]
                nr16 = nr_v[pl.ds(g * 16, 16)]
                for j in range(16):
                    nsp = jnp.full((16,), nr16[j], jnp.float32)
                    e16 = jnp.full((16,), g * 16 + j, jnp.int32)
                    t16 = jnp.full((16,), dl16[j], jnp.int32)
                    for half in range(d // 16):
                        cix = iota + half * 16
                        r = plsc.load_gather(rows_v, [e16, cix])
                        plsc.addupdate_scatter(acc_v, [t16, cix], r * nsp)
                return carry2

            lax.fori_loop(0, SUP // 16, grp, 0)
            return carry

        lax.fori_loop(0, nch, chunk, 0)
        pltpu.sync_copy(acc_v.at[pl.ds(0, RNG)], out_hbm.at[pl.ds(wid * RNG, RNG)])

    return agg


_sc_agg32 = _make_sc_agg(32)
_sc_agg16 = _make_sc_agg(16)


@functools.partial(
    pl.kernel,
    out_type=jax.ShapeDtypeStruct((NI, 16), jnp.float32),
    mesh=_MESH,
    compiler_params=_SCPARAMS,
    scratch_types=[
        pltpu.VMEM((IPW,), jnp.int32),
        pltpu.VMEM((IPW, 16), jnp.float32),
        pltpu.SemaphoreType.DMA,
    ],
)
def _sc_gather(tab_hbm, idx_hbm, out_hbm, idx_v, rows_v, sem):
    c = lax.axis_index("c")
    s = lax.axis_index("s")
    wid = s * NC + c
    base = wid * IPW
    pltpu.sync_copy(idx_hbm.at[pl.ds(base, IPW)], idx_v)
    pltpu.async_copy(tab_hbm.at[idx_v], rows_v, sem).wait()
    pltpu.sync_copy(rows_v, out_hbm.at[pl.ds(base, IPW)])


def _h0_body(x_ref, w0_ref, h0_ref):
    h0_ref[...] = jnp.dot(x_ref[...], w0_ref[...],
                          preferred_element_type=jnp.float32)


_tc_h0 = pl.pallas_call(
    _h0_body,
    grid=(GB,),
    in_specs=[
        pl.BlockSpec((RB, IN_CH), lambda i: (i, 0)),
        pl.BlockSpec((IN_CH, H), lambda i: (0, 0)),
    ],
    out_specs=pl.BlockSpec((RB, H), lambda i: (i, 0)),
    out_shape=jax.ShapeDtypeStruct((N, H), jnp.float32),
)


def _layer_body(agg_ref, h_ref, dinv2_ref, b_ref, w_ref, a_ref, hn_ref):
    a = jnp.tanh((agg_ref[...] + h_ref[...] * dinv2_ref[...]) + b_ref[...])
    a_ref[...] = a
    hn_ref[...] = jnp.dot(a, w_ref[...], preferred_element_type=jnp.float32)


def _make_tc_layer(dg):
    return pl.pallas_call(
        _layer_body,
        grid=(GB,),
        in_specs=[
            pl.BlockSpec((RB, H), lambda i: (i, 0)),
            pl.BlockSpec((RB, H), lambda i: (i, 0)),
            pl.BlockSpec((RB, 1), lambda i: (i, 0)),
            pl.BlockSpec((1, H), lambda i: (0, 0)),
            pl.BlockSpec((H, dg), lambda i: (0, 0)),
        ],
        out_specs=[
            pl.BlockSpec((RB, H), lambda i: (i, 0)),
            pl.BlockSpec((RB, dg), lambda i: (i, 0)),
        ],
        out_shape=[
            jax.ShapeDtypeStruct((N, H), jnp.float32),
            jax.ShapeDtypeStruct((N, dg), jnp.float32),
        ],
    )


_tc_layer32 = _make_tc_layer(32)
_tc_layer16 = _make_tc_layer(16)


def _tc4_body(agg3_ref, h3_ref, dinv2_ref, b3_ref, a1_ref, a2_ref, a3_ref,
              wa_ref, wb_ref, wc_ref, wl_ref, keys_ref, pre_ref):
    a4 = jnp.tanh((agg3_ref[:, 0:1] + h3_ref[:, 0:1] * dinv2_ref[...])
                  + b3_ref[...])
    keys_ref[...] = a4
    pre_ref[...] = (
        jnp.dot(a1_ref[...], wa_ref[...], preferred_element_type=jnp.float32)
        + jnp.dot(a2_ref[...], wb_ref[...], preferred_element_type=jnp.float32)
        + jnp.dot(a3_ref[...], wc_ref[...], preferred_element_type=jnp.float32)
        + a4 * wl_ref[...])


_tc4 = pl.pallas_call(
    _tc4_body,
    grid=(GB,),
    in_specs=[
        pl.BlockSpec((RB, 16), lambda i: (i, 0)),
        pl.BlockSpec((RB, 16), lambda i: (i, 0)),
        pl.BlockSpec((RB, 1), lambda i: (i, 0)),
        pl.BlockSpec((1, 1), lambda i: (0, 0)),
        pl.BlockSpec((RB, H), lambda i: (i, 0)),
        pl.BlockSpec((RB, H), lambda i: (i, 0)),
        pl.BlockSpec((RB, H), lambda i: (i, 0)),
        pl.BlockSpec((H, 16), lambda i: (0, 0)),
        pl.BlockSpec((H, 16), lambda i: (0, 0)),
        pl.BlockSpec((H, 16), lambda i: (0, 0)),
        pl.BlockSpec((1, 16), lambda i: (0, 0)),
    ],
    out_specs=[
        pl.BlockSpec((RB, 1), lambda i: (i, 0)),
        pl.BlockSpec((RB, 16), lambda i: (i, 0)),
    ],
    out_shape=[
        jax.ShapeDtypeStruct((N, 1), jnp.float32),
        jax.ShapeDtypeStruct((N, 16), jnp.float32),
    ],
)


def _topk_body(keys_ref, batch_ref, idx_ref, kscr):
    neg = jnp.float32(-jnp.inf)
    bio = lax.broadcasted_iota(jnp.int32, (B, NB), 0)
    cio = lax.broadcasted_iota(jnp.int32, (B, NB), 1)
    bt = jnp.broadcast_to(batch_ref[...], (B, NB))
    kv = jnp.broadcast_to(keys_ref[...], (B, NB))
    kscr[...] = jnp.where(bt == bio, kv, neg)
    kio = lax.broadcasted_iota(jnp.int32, (B, 32), 1)

    def it(k, carry):
        vals, idxs = carry
        kc = kscr[...]
        m = jnp.max(kc, axis=1, keepdims=True)
        idx = jnp.min(jnp.where(kc == m, cio, NB), axis=1, keepdims=True)
        kscr[...] = jnp.where(cio == idx, neg, kc)
        vals = jnp.where(kio == k, m, vals)
        idxs = jnp.where(kio == k, idx, idxs)
        return vals, idxs

    vals, idxs = lax.fori_loop(
        0, K, it,
        (jnp.full((B, 32), neg, jnp.float32), jnp.zeros((B, 32), jnp.int32)))
    idx_ref[...] = jnp.where(vals > neg, idxs, N)


_tc_topk = pl.pallas_call(
    _topk_body,
    in_specs=[
        pl.BlockSpec((1, NB), lambda: (0, 0)),
        pl.BlockSpec((1, NB), lambda: (0, 0)),
    ],
    out_specs=pl.BlockSpec((B, 32), lambda: (0, 0)),
    out_shape=jax.ShapeDtypeStruct((B, 32), jnp.int32),
    scratch_shapes=[pltpu.VMEM((B, NB), jnp.float32)],
)


def _readout_body(z0_ref, c1b_ref, se_ref, so_ref, f_ref, b2_ref,
                  m1_ref, mb1_ref, m2_ref, mb2_ref, out_ref):
    z = jnp.maximum(z0_ref[...] + c1b_ref[...], 0.0)
    a = jnp.dot(z, se_ref[...], preferred_element_type=jnp.float32)
    b = jnp.dot(z, so_ref[...], preferred_element_type=jnp.float32)
    m = jnp.maximum(a, b)
    y = jnp.maximum(
        jnp.dot(m, f_ref[...], preferred_element_type=jnp.float32)
        + b2_ref[...], 0.0)
    h = jnp.maximum(
        jnp.dot(y, m1_ref[...], preferred_element_type=jnp.float32)
        + mb1_ref[...], 0.0)
    out_ref[...] = (jnp.dot(h, m2_ref[...], preferred_element_type=jnp.float32)
                    + mb2_ref[...])


_tc_readout = pl.pallas_call(
    _readout_body,
    out_shape=jax.ShapeDtypeStruct((B, 5), jnp.float32),
)

_SE_NP = np.zeros((480, 240), np.float32)
_SO_NP = np.zeros((480, 240), np.float32)
for _p in range(15):
    for _i in range(16):
        _SE_NP[(2 * _p) * 16 + _i, _p * 16 + _i] = 1.0
        _SO_NP[(2 * _p + 1) * 16 + _i, _p * 16 + _i] = 1.0

_POS = np.arange(15)[:, None]
_PP = np.arange(11)[None, :]
_TT = _POS - _PP
_VALID = ((_TT >= 0) & (_TT < 5)).astype(np.float32)
_TCIDX = np.clip(_TT, 0, 4)


def kernel(x, edge_index, batch, W0, b0, W1, b1, W2, b2, W3, b3,
           conv1_w, conv1_b, conv2_w, conv2_b, mlp_w1, mlp_b1, mlp_w2, mlp_b2):
    f32 = jnp.float32
    src = edge_index[0]
    dst = edge_index[1]
    dstp = jnp.concatenate([dst, jnp.full((EP - E,), N, jnp.int32)])
    ones_c = jnp.ones((CH, 16), f32)
    zeros16 = jnp.zeros((NPAD, 16), f32)
    zacc32 = jnp.zeros((ACC, 32), f32)
    zacc16 = jnp.zeros((ACC, 16), f32)

    def pad_rows(g):
        return jnp.concatenate([g, jnp.zeros((NPAD - N, g.shape[1]), f32)])

    degp = _sc_deg(dstp, ones_c, zeros16).reshape(NC, NPAD, 16)
    deg = (degp[0, :N, 0] + degp[1, :N, 0]) + 1.0
    dinv = jnp.where(deg > 0, 1.0 / jnp.sqrt(deg), 0.0)
    dinv2 = (dinv * dinv).reshape(N, 1)
    dinv_p = jnp.concatenate([dinv, jnp.zeros((NPAD - N,), f32)])

    srcg, dloc, nrm, cnts = _sc_setup(src, dst, dinv_p)

    h0 = _tc_h0(x, W0)
    agg0 = _sc_agg32(pad_rows(h0), srcg, dloc, nrm, cnts, zacc32)
    a1, h1 = _tc_layer32(agg0[:N], h0, dinv2, b0.reshape(1, H), W1)
    agg1 = _sc_agg32(pad_rows(h1), srcg, dloc, nrm, cnts, zacc32)
    a2, h2 = _tc_layer32(agg1[:N], h1, dinv2, b1.reshape(1, H), W2)
    agg2 = _sc_agg32(pad_rows(h2), srcg, dloc, nrm, cnts, zacc32)
    w3p = jnp.concatenate([W3, jnp.zeros((H, 15), f32)], axis=1)
    a3, h3 = _tc_layer16(agg2[:N], h2, dinv2, b2.reshape(1, H), w3p)
    agg3 = _sc_agg16(pad_rows(h3), srcg, dloc, nrm, cnts, zacc16)

    wc1 = conv1_w[:, 0, :]
    wa = wc1[:, 0:32].T
    wb = wc1[:, 32:64].T
    wcc = wc1[:, 64:96].T
    wl = wc1[:, 96:97].T
    keys, pre = _tc4(agg3[:N], h3, dinv2, b3.reshape(1, 1), a1, a2, a3,
                     wa, wb, wcc, wl)

    keysp = jnp.concatenate([keys[:, 0], jnp.zeros((NB - N,), f32)])
    batchp = jnp.concatenate([batch, jnp.full((NB - N,), B, jnp.int32)])
    idx = _tc_topk(keysp.reshape(1, NB), batchp.reshape(1, NB))
    idx_flat = idx[:, :K].reshape(B * K)
    idxp = jnp.concatenate(
        [idx_flat, jnp.full((NI - B * K,), N, jnp.int32)])
    rows = _sc_gather(pad_rows(pre), idxp)
    z0 = rows[: B * K].reshape(B, K * 16)

    fmat = (conv2_w[:, :, _TCIDX] * jnp.asarray(_VALID)[None, None])
    fmat = fmat.transpose(2, 1, 0, 3).reshape(240, 352)
    c1b = jnp.tile(conv1_b, K).reshape(1, K * 16)
    b2t = jnp.repeat(conv2_b, 11).reshape(1, 352)
    out = _tc_readout(z0, c1b, jnp.asarray(_SE_NP), jnp.asarray(_SO_NP),
                      fmat, b2t, mlp_w1, mlp_b1.reshape(1, 128),
                      mlp_w2, mlp_b2.reshape(1, 5))
    return out
